# Initial kernel scaffold; baseline (speedup 1.0000x reference)
import jax
import jax.numpy as jnp
from jax.experimental import pallas as pl


def kernel(x, position_ids):
    raise NotImplementedError("write your pallas kernel here")


# speedup vs baseline: 1.1422x; 1.1422x over previous
import functools

import jax
import jax.numpy as jnp
from jax import lax
from jax.experimental import pallas as pl
from jax.experimental.pallas import tpu as pltpu
from jax.experimental.pallas import tpu_sc as plsc

DIM = 64
MAX_POS = 8192
THETA = 10000.0
SEQ = 8192

NC = 2
NS = 16
NW = NC * NS
BPW = SEQ // NW
CHUNK = 128
NCH = BPW // CHUNK


def _tables():
    inv_freq = 1.0 / (THETA ** (jnp.arange(0, DIM, 2, dtype=jnp.float32) / DIM))
    t = jnp.arange(MAX_POS, dtype=jnp.float32)
    freqs = t[:, None] * inv_freq[None, :]
    emb = jnp.concatenate((freqs, freqs), axis=-1)
    return jnp.cos(emb), jnp.sin(emb)


_mesh = plsc.VectorSubcoreMesh(core_axis_name="c", subcore_axis_name="s")


@functools.partial(
    pl.kernel,
    mesh=_mesh,
    out_type=(
        jax.ShapeDtypeStruct((SEQ, DIM), jnp.float32),
        jax.ShapeDtypeStruct((SEQ, DIM), jnp.float32),
    ),
    scratch_types=[
        pltpu.VMEM((NCH, CHUNK), jnp.int32),
        pltpu.VMEM((BPW, DIM), jnp.float32),
        pltpu.VMEM((BPW, DIM), jnp.float32),
        pltpu.SemaphoreType.DMA,
    ],
    compiler_params=pltpu.CompilerParams(use_tc_tiling_on_sc=False),
)
def _rope_gather(cos_hbm, sin_hbm, idx_hbm, cos_out, sin_out,
                 idx_v, cos_v, sin_v, sem):
    wid = lax.axis_index("s") * NC + lax.axis_index("c")
    base = wid * BPW
    pltpu.sync_copy(idx_hbm.at[pl.ds(wid * NCH, NCH)], idx_v)
    copies = []
    for j in range(NCH):
        copies.append(pltpu.async_copy(
            cos_hbm.at[idx_v.at[j]], cos_v.at[pl.ds(j * CHUNK, CHUNK)], sem))
        copies.append(pltpu.async_copy(
            sin_hbm.at[idx_v.at[j]], sin_v.at[pl.ds(j * CHUNK, CHUNK)], sem))
    for c in copies:
        c.wait()
    pltpu.sync_copy(cos_v, cos_out.at[pl.ds(base, BPW)])
    pltpu.sync_copy(sin_v, sin_out.at[pl.ds(base, BPW)])


def kernel(x, position_ids):
    cos_tab, sin_tab = _tables()
    idx = position_ids.reshape(NW * NCH, CHUNK).astype(jnp.int32)
    cos, sin = _rope_gather(cos_tab, sin_tab, idx)
    cos = cos.reshape(1, 1, SEQ, DIM).astype(x.dtype)
    sin = sin.reshape(1, 1, SEQ, DIM).astype(x.dtype)
    return (cos, sin)


# speedup vs baseline: 1.6570x; 1.4507x over previous
import functools

import jax
import jax.numpy as jnp
import numpy as np
from jax import lax
from jax.experimental import pallas as pl
from jax.experimental.pallas import tpu as pltpu
from jax.experimental.pallas import tpu_sc as plsc

DIM = 64
MAX_POS = 8192
THETA = 10000.0
SEQ = 8192

NC = 2
NS = 16
NW = NC * NS
BPW = SEQ // NW
CHUNK = 128
NCH = BPW // CHUNK


def _tables():
    inv_freq = 1.0 / (THETA ** (np.arange(0, DIM, 2, dtype=np.float32) / DIM))
    t = np.arange(MAX_POS, dtype=np.float32)
    freqs = (t[:, None] * inv_freq[None, :]).astype(np.float32)
    emb = np.concatenate((freqs, freqs), axis=-1)
    return np.cos(emb).astype(np.float32), np.sin(emb).astype(np.float32)


_COS_TAB, _SIN_TAB = _tables()


_mesh = plsc.VectorSubcoreMesh(core_axis_name="c", subcore_axis_name="s")


@functools.partial(
    pl.kernel,
    mesh=_mesh,
    out_type=(
        jax.ShapeDtypeStruct((SEQ, DIM), jnp.float32),
        jax.ShapeDtypeStruct((SEQ, DIM), jnp.float32),
    ),
    scratch_types=[
        pltpu.VMEM((NCH, CHUNK), jnp.int32),
        pltpu.VMEM((BPW, DIM), jnp.float32),
        pltpu.VMEM((BPW, DIM), jnp.float32),
        pltpu.SemaphoreType.DMA,
    ],
    compiler_params=pltpu.CompilerParams(use_tc_tiling_on_sc=False),
)
def _rope_gather(cos_hbm, sin_hbm, idx_hbm, cos_out, sin_out,
                 idx_v, cos_v, sin_v, sem):
    wid = lax.axis_index("s") * NC + lax.axis_index("c")
    base = wid * BPW
    pltpu.sync_copy(idx_hbm.at[pl.ds(wid * NCH, NCH)], idx_v)
    copies = []
    for j in range(NCH):
        copies.append(pltpu.async_copy(
            cos_hbm.at[idx_v.at[j]], cos_v.at[pl.ds(j * CHUNK, CHUNK)], sem))
        copies.append(pltpu.async_copy(
            sin_hbm.at[idx_v.at[j]], sin_v.at[pl.ds(j * CHUNK, CHUNK)], sem))
    for c in copies:
        c.wait()
    pltpu.sync_copy(cos_v, cos_out.at[pl.ds(base, BPW)])
    pltpu.sync_copy(sin_v, sin_out.at[pl.ds(base, BPW)])


def kernel(x, position_ids):
    cos_tab = jnp.asarray(_COS_TAB)
    sin_tab = jnp.asarray(_SIN_TAB)
    idx = position_ids.reshape(NW * NCH, CHUNK).astype(jnp.int32)
    cos, sin = _rope_gather(cos_tab, sin_tab, idx)
    cos = cos.reshape(1, 1, SEQ, DIM).astype(x.dtype)
    sin = sin.reshape(1, 1, SEQ, DIM).astype(x.dtype)
    return (cos, sin)


# speedup vs baseline: 1.6623x; 1.0032x over previous
import functools

import jax
import jax.numpy as jnp
import numpy as np
from jax import lax
from jax.experimental import pallas as pl
from jax.experimental.pallas import tpu as pltpu
from jax.experimental.pallas import tpu_sc as plsc

DIM = 64
MAX_POS = 8192
THETA = 10000.0
SEQ = 8192

NC = 2
NS = 16
NW = NC * NS
BPW = SEQ // NW
CHUNK = 128
NCH = BPW // CHUNK


def _tables():
    inv_freq = 1.0 / (THETA ** (np.arange(0, DIM, 2, dtype=np.float32) / DIM))
    t = np.arange(MAX_POS, dtype=np.float32)
    freqs = (t[:, None] * inv_freq[None, :]).astype(np.float32)
    emb = np.concatenate((freqs, freqs), axis=-1)
    return np.cos(emb).astype(np.float32), np.sin(emb).astype(np.float32)


_COS_TAB, _SIN_TAB = _tables()

_mesh = plsc.VectorSubcoreMesh(core_axis_name="c", subcore_axis_name="s")


@functools.partial(
    pl.kernel,
    mesh=_mesh,
    out_type=(
        jax.ShapeDtypeStruct((1, 1, SEQ, DIM), jnp.float32),
        jax.ShapeDtypeStruct((1, 1, SEQ, DIM), jnp.float32),
    ),
    scratch_types=[
        pltpu.VMEM((BPW,), jnp.int32),
        pltpu.VMEM((BPW, DIM), jnp.float32),
        pltpu.VMEM((BPW, DIM), jnp.float32),
        pltpu.SemaphoreType.DMA,
    ],
    compiler_params=pltpu.CompilerParams(use_tc_tiling_on_sc=False),
)
def _rope_gather(cos_hbm, sin_hbm, idx_hbm, cos_out, sin_out,
                 idx_v, cos_v, sin_v, sem):
    wid = lax.axis_index("s") * NC + lax.axis_index("c")
    base = wid * BPW
    pltpu.sync_copy(idx_hbm.at[pl.ds(base, BPW)], idx_v)
    copies = []
    for j in range(NCH):
        idx_sl = idx_v.at[pl.ds(j * CHUNK, CHUNK)]
        copies.append(pltpu.async_copy(
            cos_hbm.at[idx_sl], cos_v.at[pl.ds(j * CHUNK, CHUNK)], sem))
        copies.append(pltpu.async_copy(
            sin_hbm.at[idx_sl], sin_v.at[pl.ds(j * CHUNK, CHUNK)], sem))
    for c in copies:
        c.wait()
    pltpu.sync_copy(cos_v, cos_out.at[0, 0, pl.ds(base, BPW)])
    pltpu.sync_copy(sin_v, sin_out.at[0, 0, pl.ds(base, BPW)])


def kernel(x, position_ids):
    cos_tab = jnp.asarray(_COS_TAB)
    sin_tab = jnp.asarray(_SIN_TAB)
    idx = position_ids.reshape(SEQ).astype(jnp.int32)
    cos, sin = _rope_gather(cos_tab, sin_tab, idx)
    return (cos.astype(x.dtype), sin.astype(x.dtype))
